# baseline (speedup 1.0000x reference)
import functools

import jax
import jax.numpy as jnp
from jax import lax
from jax.experimental import pallas as pl
from jax.experimental.pallas import tpu as pltpu
from jax.experimental.pallas import tpu_sc as plsc

C, H, W = 768, 32, 32
HW = H * W
B = 64

try:
    _info = plsc.get_sparse_core_info()
    _NC, _NS = _info.num_cores, _info.num_subcores
except Exception:
    _NC, _NS = 2, 16
_NW = _NC * _NS
_ROWS_PER_W = HW // _NW


def _sc_gather(table, idx):
    mesh = plsc.VectorSubcoreMesh(core_axis_name="c", subcore_axis_name="s")

    @functools.partial(
        pl.kernel,
        mesh=mesh,
        out_type=jax.ShapeDtypeStruct((HW, C), jnp.float32),
        scratch_types=[
            pltpu.VMEM((_ROWS_PER_W,), jnp.int32),
            pltpu.VMEM((_ROWS_PER_W, C), jnp.float32),
            pltpu.SemaphoreType.DMA,
        ],
    )
    def k(table_hbm, idx_hbm, out_hbm, idx_v, rows_v, sem):
        wid = lax.axis_index("s") * _NC + lax.axis_index("c")
        base = wid * _ROWS_PER_W
        pltpu.sync_copy(idx_hbm.at[pl.ds(base, _ROWS_PER_W)], idx_v)
        pltpu.async_copy(table_hbm.at[idx_v], rows_v, sem).wait()
        pltpu.sync_copy(rows_v, out_hbm.at[pl.ds(base, _ROWS_PER_W)])

    return k(table, idx)


_NBUF = 4


def _add_body(x_hbm, pe_hbm, o_hbm, *scratch):
    inbs = scratch[:_NBUF]
    outbs = scratch[_NBUF:2 * _NBUF]
    peb = scratch[2 * _NBUF]
    insems = scratch[2 * _NBUF + 1:3 * _NBUF + 1]
    outsems = scratch[3 * _NBUF + 1:4 * _NBUF + 1]
    pesem = scratch[4 * _NBUF + 1]
    pe_cp = pltpu.make_async_copy(pe_hbm, peb, pesem)
    pe_cp.start()
    for j in range(_NBUF):
        pltpu.make_async_copy(x_hbm.at[j], inbs[j], insems[j]).start()
    pe_cp.wait()

    def outer(i, carry):
        for j in range(_NBUF):
            b = i * _NBUF + j
            pltpu.make_async_copy(x_hbm.at[b], inbs[j], insems[j]).wait()

            @pl.when(i > 0)
            def _wait_flush():
                pltpu.make_async_copy(outbs[j], o_hbm.at[b], outsems[j]).wait()

            outbs[j][:, :] = inbs[j][:, :] + peb[:, :]
            pltpu.make_async_copy(outbs[j], o_hbm.at[b], outsems[j]).start()
            nb = b + _NBUF

            @pl.when(nb < B)
            def _fetch_next():
                pltpu.make_async_copy(x_hbm.at[nb], inbs[j], insems[j]).start()

        return carry

    jax.lax.fori_loop(0, B // _NBUF, outer, 0)
    for j in range(_NBUF):
        pltpu.make_async_copy(outbs[j], o_hbm.at[B - _NBUF + j], outsems[j]).wait()


def _tc_add(x3, pe2):
    return pl.pallas_call(
        _add_body,
        in_specs=[
            pl.BlockSpec(memory_space=pl.ANY),
            pl.BlockSpec(memory_space=pl.ANY),
        ],
        out_specs=pl.BlockSpec(memory_space=pl.ANY),
        out_shape=jax.ShapeDtypeStruct((B, C, HW), jnp.float32),
        scratch_shapes=(
            [pltpu.VMEM((C, HW), jnp.float32)] * (2 * _NBUF)
            + [pltpu.VMEM((C, HW), jnp.float32)]
            + [pltpu.SemaphoreType.DMA] * (2 * _NBUF + 1)
        ),
        compiler_params=pltpu.CompilerParams(
            vmem_limit_bytes=100 * 1024 * 1024,
        ),
    )(x3, pe2)


def kernel(x, table, position_ids):
    idx = position_ids.astype(jnp.int32)
    pe = _sc_gather(table, idx)
    pe2 = pe.reshape(C, HW)
    x3 = x.reshape(B, C, HW)
    out3 = _tc_add(x3, pe2)
    return out3.reshape(B, C, H, W)


# speedup vs baseline: 3.2303x; 3.2303x over previous
import functools

import jax
import jax.numpy as jnp
from jax import lax
from jax.experimental import pallas as pl
from jax.experimental.pallas import tpu as pltpu
from jax.experimental.pallas import tpu_sc as plsc

C, H, W = 768, 32, 32
HW = H * W
B = 64

try:
    _info = plsc.get_sparse_core_info()
    _NC, _NS = _info.num_cores, _info.num_subcores
except Exception:
    _NC, _NS = 2, 16
_NW = _NC * _NS
_ROWS_PER_W = HW // _NW


def _sc_gather(table, idx):
    mesh = plsc.VectorSubcoreMesh(core_axis_name="c", subcore_axis_name="s")

    @functools.partial(
        pl.kernel,
        mesh=mesh,
        out_type=jax.ShapeDtypeStruct((HW, C), jnp.float32),
        scratch_types=[
            pltpu.VMEM((_ROWS_PER_W,), jnp.int32),
            pltpu.VMEM((_ROWS_PER_W, C), jnp.float32),
            pltpu.SemaphoreType.DMA,
        ],
    )
    def k(table_hbm, idx_hbm, out_hbm, idx_v, rows_v, sem):
        wid = lax.axis_index("s") * _NC + lax.axis_index("c")
        base = wid * _ROWS_PER_W
        pltpu.sync_copy(idx_hbm.at[pl.ds(base, _ROWS_PER_W)], idx_v)
        pltpu.async_copy(table_hbm.at[idx_v], rows_v, sem).wait()
        pltpu.sync_copy(rows_v, out_hbm.at[pl.ds(base, _ROWS_PER_W)])

    return k(table, idx)


_NBUF = 4


def _add_body(x_hbm, pe_hbm, o_hbm, *scratch):
    inbs = scratch[:_NBUF]
    outbs = scratch[_NBUF:2 * _NBUF]
    peb = scratch[2 * _NBUF]
    insems = scratch[2 * _NBUF + 1:3 * _NBUF + 1]
    outsems = scratch[3 * _NBUF + 1:4 * _NBUF + 1]
    pesem = scratch[4 * _NBUF + 1]
    pe_cp = pltpu.make_async_copy(pe_hbm, peb, pesem)
    pe_cp.start()
    for j in range(_NBUF):
        pltpu.make_async_copy(x_hbm.at[j], inbs[j], insems[j]).start()
    pe_cp.wait()

    def outer(i, carry):
        for j in range(_NBUF):
            b = i * _NBUF + j
            pltpu.make_async_copy(x_hbm.at[b], inbs[j], insems[j]).wait()

            @pl.when(i > 0)
            def _wait_flush():
                pltpu.make_async_copy(outbs[j], o_hbm.at[b], outsems[j]).wait()

            outbs[j][:, :] = inbs[j][:, :] + peb[:, :]
            pltpu.make_async_copy(outbs[j], o_hbm.at[b], outsems[j]).start()
            nb = b + _NBUF

            @pl.when(nb < B)
            def _fetch_next():
                pltpu.make_async_copy(x_hbm.at[nb], inbs[j], insems[j]).start()

        return carry

    jax.lax.fori_loop(0, B // _NBUF, outer, 0)
    for j in range(_NBUF):
        pltpu.make_async_copy(outbs[j], o_hbm.at[B - _NBUF + j], outsems[j]).wait()


def _tc_add(x3, pe2):
    return pl.pallas_call(
        _add_body,
        in_specs=[
            pl.BlockSpec(memory_space=pl.ANY),
            pl.BlockSpec(memory_space=pl.ANY),
        ],
        out_specs=pl.BlockSpec(memory_space=pl.ANY),
        out_shape=jax.ShapeDtypeStruct((B, HW, C), jnp.float32),
        scratch_shapes=(
            [pltpu.VMEM((HW, C), jnp.float32)] * (2 * _NBUF + 1)
            + [pltpu.SemaphoreType.DMA] * (2 * _NBUF + 1)
        ),
        compiler_params=pltpu.CompilerParams(
            vmem_limit_bytes=100 * 1024 * 1024,
        ),
    )(x3, pe2)


def kernel(x, table, position_ids):
    idx = position_ids.astype(jnp.int32)
    pe = _sc_gather(table, idx)
    pe_t = pe.reshape(C, HW).T
    x4 = x.transpose(0, 2, 3, 1).reshape(B, HW, C)
    out4 = _tc_add(x4, pe_t)
    return out4.reshape(B, H, W, C).transpose(0, 3, 1, 2)
